# baseline (speedup 1.0000x reference)
import functools

import jax
import jax.numpy as jnp
from jax import lax
from jax.experimental import pallas as pl
from jax.experimental.pallas import tpu as pltpu
from jax.experimental.pallas import tpu_sc as plsc

_B, _S = 4, 8192
_T = _B * _S
_TB = 2048
_NB = _T // _TB
_EPS = 1e-8

_NC, _NS = 2, 16
_NW = _NC * _NS
_PER_W = _T // _NW
_CHUNK = 128
_NCHUNK = _PER_W // _CHUNK


def _softplus(x):
    return jnp.maximum(x, 0.0) + jnp.log1p(jnp.exp(-jnp.abs(x)))


def _sc_gather_kernel(tabp, tabd, tabv,
                      ixtp, ix1p, ixtd, ix1d, ixtv, ix1v,
                      oxtp, ox1p, oxtd, ox1d, oxtv, ox1v,
                      xi_v, idx_v, rows_v, sem):
    wid = lax.axis_index("s") * _NC + lax.axis_index("c")

    def gather_pairs(tab, idx_hbm, out_hbm, vr):
        def chunk(c, _):
            base = wid * _PER_W + c * _CHUNK
            pltpu.sync_copy(idx_hbm.at[pl.ds(base, _CHUNK)], xi_v)
            for j in range(_CHUNK // 16):
                xi = xi_v[pl.ds(j * 16, 16)]
                tok = base + j * 16 + lax.iota(jnp.int32, 16)
                idx_v[pl.ds(j * 16, 16)] = tok * vr + (xi >> 4)
            pltpu.async_copy(tab.at[idx_v], rows_v, sem).wait()
            pltpu.sync_copy(rows_v, out_hbm.at[pl.ds(base, _CHUNK)])
            return ()

        for c in range(_NCHUNK):
            chunk(c, ())

    gather_pairs(tabp, ixtp, oxtp, 8)
    gather_pairs(tabp, ix1p, ox1p, 8)
    gather_pairs(tabd, ixtd, oxtd, 32)
    gather_pairs(tabd, ix1d, ox1d, 32)
    gather_pairs(tabv, ixtv, oxtv, 8)
    gather_pairs(tabv, ix1v, ox1v, 8)


def _sc_gather(tabp, tabd, tabv, ixtp, ix1p, ixtd, ix1d, ixtv, ix1v):
    mesh = plsc.VectorSubcoreMesh(core_axis_name="c", subcore_axis_name="s")
    g16 = jax.ShapeDtypeStruct((_T, 16), jnp.float32)
    return pl.kernel(
        _sc_gather_kernel,
        out_type=[g16] * 6,
        mesh=mesh,
        scratch_types=[
            pltpu.VMEM((_CHUNK,), jnp.int32),
            pltpu.VMEM((_CHUNK,), jnp.int32),
            pltpu.VMEM((_CHUNK, 16), jnp.float32),
            pltpu.SemaphoreType.DMA,
        ],
        compiler_params=pltpu.CompilerParams(use_tc_tiling_on_sc=False),
    )(tabp, tabd, tabv, ixtp, ix1p, ixtd, ix1d, ixtv, ix1v)


def _attr_loss(l, g_xt, g_x1, xt_row, x0_row, x1_row, ll_row, tl_row,
               disallow_zero):
    TB, V = l.shape
    m = jnp.max(l, axis=1, keepdims=True)
    s = jnp.exp(l - m)
    zfull_c = jnp.sum(s, axis=1, keepdims=True)
    e0_c = s[:, 0:1]

    off_xt_c = jnp.transpose(xt_row & 15)
    off_x1_c = jnp.transpose(x1_row & 15)
    iota16 = lax.broadcasted_iota(jnp.int32, (TB, 16), 1)
    e_xt_c = jnp.sum(jnp.where(iota16 == off_xt_c, jnp.exp(g_xt - m), 0.0),
                     axis=1, keepdims=True)
    e_x1_c = jnp.sum(jnp.where(iota16 == off_x1_c, jnp.exp(g_x1 - m), 0.0),
                     axis=1, keepdims=True)

    zfull = jnp.transpose(zfull_c)
    e0 = jnp.transpose(e0_c)
    e_xt = jnp.transpose(e_xt_c)
    e_x1 = jnp.transpose(e_x1_c)

    if disallow_zero:
        z = zfull - e_xt - jnp.where(xt_row == 0, 0.0, e0)
    else:
        z = zfull - e_xt
    p1 = e_x1 / z
    if disallow_zero:
        p1 = jnp.where(x1_row == 0, 0.0, p1)

    sm = (xt_row == x0_row) & (x1_row != xt_row)
    coef = jnp.where(sm, tl_row, 0.0)
    lam = _softplus(ll_row)
    tok = lam - coef * jnp.log(lam * p1 + _EPS)
    return jnp.sum(tok)


def _loss_kernel(lp_ref, ld_ref, lv_ref,
                 gxtp_ref, gx1p_ref, gxtd_ref, gx1d_ref, gxtv_ref, gx1v_ref,
                 xtp_ref, x0p_ref, x1p_ref, llp_ref,
                 xtd_ref, x0d_ref, x1d_ref, lld_ref,
                 xtv_ref, x0v_ref, x1v_ref, llv_ref,
                 tl_ref, out_ref):
    i = pl.program_id(0)

    @pl.when(i == 0)
    def _():
        out_ref[...] = jnp.zeros_like(out_ref)

    tl_row = tl_ref[0]
    acc = _attr_loss(lp_ref[...], gxtp_ref[...], gx1p_ref[...],
                     xtp_ref[0], x0p_ref[0], x1p_ref[0], llp_ref[0],
                     tl_row, False)
    acc += _attr_loss(ld_ref[...], gxtd_ref[...], gx1d_ref[...],
                      xtd_ref[0], x0d_ref[0], x1d_ref[0], lld_ref[0],
                      tl_row, True)
    acc += _attr_loss(lv_ref[...], gxtv_ref[...], gx1v_ref[...],
                      xtv_ref[0], x0v_ref[0], x1v_ref[0], llv_ref[0],
                      tl_row, True)
    out_ref[...] += jnp.full((1, 1), acc)


def kernel(token_logits_pitch, token_logits_duration, token_logits_velocity,
           log_lambda_pitch, log_lambda_duration, log_lambda_velocity,
           x_t_pitch, x_t_duration, x_t_velocity,
           x_0_pitch, x_0_duration, x_0_velocity,
           x_1_pitch, x_1_duration, x_1_velocity,
           t):
    tl = 1.0 / jnp.clip(1.0 - t, 1e-8, None)
    tl_full = jnp.broadcast_to(tl[:, None], (_B, _S))

    lp = token_logits_pitch.reshape(_T, 128)
    ld = token_logits_duration.reshape(_T, 512)
    lv = token_logits_velocity.reshape(_T, 128)

    gxtp, gx1p, gxtd, gx1d, gxtv, gx1v = _sc_gather(
        lp.reshape(-1, 16), ld.reshape(-1, 16), lv.reshape(-1, 16),
        x_t_pitch.reshape(_T), x_1_pitch.reshape(_T),
        x_t_duration.reshape(_T), x_1_duration.reshape(_T),
        x_t_velocity.reshape(_T), x_1_velocity.reshape(_T))

    def meta(x):
        return x.reshape(_NB, 1, _TB)

    args = [
        lp, ld, lv,
        gxtp, gx1p, gxtd, gx1d, gxtv, gx1v,
        meta(x_t_pitch), meta(x_0_pitch), meta(x_1_pitch), meta(log_lambda_pitch),
        meta(x_t_duration), meta(x_0_duration), meta(x_1_duration), meta(log_lambda_duration),
        meta(x_t_velocity), meta(x_0_velocity), meta(x_1_velocity), meta(log_lambda_velocity),
        meta(tl_full),
    ]

    def logits_spec(v):
        return pl.BlockSpec((_TB, v), lambda i: (i, 0))

    g_spec = pl.BlockSpec((_TB, 16), lambda i: (i, 0))
    meta_spec = pl.BlockSpec((1, 1, _TB), lambda i: (i, 0, 0))

    out = pl.pallas_call(
        _loss_kernel,
        grid=(_NB,),
        in_specs=[logits_spec(128), logits_spec(512), logits_spec(128)]
                 + [g_spec] * 6
                 + [meta_spec] * 13,
        out_specs=pl.BlockSpec((1, 1), lambda i: (0, 0)),
        out_shape=jax.ShapeDtypeStruct((1, 1), jnp.float32),
    )(*args)
    return out[0, 0] / _T


# speedup vs baseline: 1.4022x; 1.4022x over previous
import jax
import jax.numpy as jnp
from jax import lax
from jax.experimental import pallas as pl
from jax.experimental.pallas import tpu as pltpu
from jax.experimental.pallas import tpu_sc as plsc

_B, _S = 4, 8192
_T = _B * _S
_TB = 2048
_NB = _T // _TB
_EPS = 1e-8

_NC, _NS = 2, 16
_NW = _NC * _NS
_PER_W = _T // _NW
_CHUNK = 512
_NCHUNK = _PER_W // _CHUNK


def _softplus(x):
    return jnp.maximum(x, 0.0) + jnp.log1p(jnp.exp(-jnp.abs(x)))


def _sc_gather_kernel(tabp, tabd, tabv, ix1p, ix1d, ix1v,
                      ox1p, ox1d, ox1v, xi_v, idx_v, rows_v, sem):
    wid = lax.axis_index("s") * _NC + lax.axis_index("c")

    def gather(tab, idx_hbm, out_hbm, vr):
        for c in range(_NCHUNK):
            base = wid * _PER_W + c * _CHUNK
            pltpu.sync_copy(idx_hbm.at[pl.ds(base, _CHUNK)], xi_v)
            for j in range(_CHUNK // 16):
                xi = xi_v[pl.ds(j * 16, 16)]
                tok = base + j * 16 + lax.iota(jnp.int32, 16)
                idx_v[pl.ds(j * 16, 16)] = tok * vr + (xi >> 4)
            pltpu.async_copy(tab.at[idx_v], rows_v, sem).wait()
            pltpu.sync_copy(rows_v, out_hbm.at[pl.ds(base, _CHUNK)])

    gather(tabp, ix1p, ox1p, 8)
    gather(tabd, ix1d, ox1d, 32)
    gather(tabv, ix1v, ox1v, 8)


def _sc_gather(tabp, tabd, tabv, ix1p, ix1d, ix1v):
    mesh = plsc.VectorSubcoreMesh(core_axis_name="c", subcore_axis_name="s")
    g16 = jax.ShapeDtypeStruct((_T, 16), jnp.float32)
    return pl.kernel(
        _sc_gather_kernel,
        out_type=[g16] * 3,
        mesh=mesh,
        scratch_types=[
            pltpu.VMEM((_CHUNK,), jnp.int32),
            pltpu.VMEM((_CHUNK,), jnp.int32),
            pltpu.VMEM((_CHUNK, 16), jnp.float32),
            pltpu.SemaphoreType.DMA,
        ],
        compiler_params=pltpu.CompilerParams(use_tc_tiling_on_sc=False),
    )(tabp, tabd, tabv, ix1p, ix1d, ix1v)


def _attr_stats(l, xt_row, disallow_zero):
    TB, V = l.shape
    m = jnp.max(l, axis=1, keepdims=True)
    s = jnp.exp(l - m)
    zfull = jnp.sum(s, axis=1, keepdims=True)
    xt_c = jnp.transpose(xt_row)
    iota = lax.broadcasted_iota(jnp.int32, (TB, V), 1)
    e_xt = jnp.sum(jnp.where(iota == xt_c, s, 0.0), axis=1, keepdims=True)
    if disallow_zero:
        e0 = s[:, 0:1]
        z = zfull - e_xt - jnp.where(xt_c == 0, 0.0, e0)
    else:
        z = zfull - e_xt
    return m, z


def _stats_kernel(lp_ref, ld_ref, lv_ref, xtp_ref, xtd_ref, xtv_ref,
                  mp_ref, zp_ref, md_ref, zd_ref, mv_ref, zv_ref):
    mp_ref[...], zp_ref[...] = _attr_stats(lp_ref[...], xtp_ref[0], False)
    md_ref[...], zd_ref[...] = _attr_stats(ld_ref[...], xtd_ref[0], True)
    mv_ref[...], zv_ref[...] = _attr_stats(lv_ref[...], xtv_ref[0], True)


def _attr_loss(g_x1, m_c, z_c, xt_row, x0_row, x1_row, ll_row, tl_row,
               disallow_zero):
    TB = g_x1.shape[0]
    off_c = jnp.transpose(x1_row & 15)
    iota16 = lax.broadcasted_iota(jnp.int32, (TB, 16), 1)
    e_x1_c = jnp.sum(jnp.where(iota16 == off_c, jnp.exp(g_x1 - m_c), 0.0),
                     axis=1, keepdims=True)
    p1 = jnp.transpose(e_x1_c / z_c)
    if disallow_zero:
        p1 = jnp.where(x1_row == 0, 0.0, p1)

    sm = (xt_row == x0_row) & (x1_row != xt_row)
    coef = jnp.where(sm, tl_row, 0.0)
    lam = _softplus(ll_row)
    tok = lam - coef * jnp.log(lam * p1 + _EPS)
    return jnp.sum(tok)


def _combine_kernel(gx1p_ref, gx1d_ref, gx1v_ref,
                    mp_ref, zp_ref, md_ref, zd_ref, mv_ref, zv_ref,
                    xtp_ref, x0p_ref, x1p_ref, llp_ref,
                    xtd_ref, x0d_ref, x1d_ref, lld_ref,
                    xtv_ref, x0v_ref, x1v_ref, llv_ref,
                    tl_ref, out_ref):
    i = pl.program_id(0)

    @pl.when(i == 0)
    def _():
        out_ref[...] = jnp.zeros_like(out_ref)

    tl_row = tl_ref[0]
    acc = _attr_loss(gx1p_ref[...], mp_ref[...], zp_ref[...],
                     xtp_ref[0], x0p_ref[0], x1p_ref[0], llp_ref[0],
                     tl_row, False)
    acc += _attr_loss(gx1d_ref[...], md_ref[...], zd_ref[...],
                      xtd_ref[0], x0d_ref[0], x1d_ref[0], lld_ref[0],
                      tl_row, True)
    acc += _attr_loss(gx1v_ref[...], mv_ref[...], zv_ref[...],
                      xtv_ref[0], x0v_ref[0], x1v_ref[0], llv_ref[0],
                      tl_row, True)
    out_ref[...] += jnp.full((1, 1), acc)


def kernel(token_logits_pitch, token_logits_duration, token_logits_velocity,
           log_lambda_pitch, log_lambda_duration, log_lambda_velocity,
           x_t_pitch, x_t_duration, x_t_velocity,
           x_0_pitch, x_0_duration, x_0_velocity,
           x_1_pitch, x_1_duration, x_1_velocity,
           t):
    tl = 1.0 / jnp.clip(1.0 - t, 1e-8, None)
    tl_full = jnp.broadcast_to(tl[:, None], (_B, _S))

    lp = token_logits_pitch.reshape(_T, 128)
    ld = token_logits_duration.reshape(_T, 512)
    lv = token_logits_velocity.reshape(_T, 128)

    gx1p, gx1d, gx1v = _sc_gather(
        lp.reshape(-1, 16), ld.reshape(-1, 16), lv.reshape(-1, 16),
        x_1_pitch.reshape(_T), x_1_duration.reshape(_T),
        x_1_velocity.reshape(_T))

    def meta(x):
        return x.reshape(_NB, 1, _TB)

    meta_spec = pl.BlockSpec((1, 1, _TB), lambda i: (i, 0, 0))
    col_spec = pl.BlockSpec((_TB, 1), lambda i: (i, 0))
    col_shape = jax.ShapeDtypeStruct((_T, 1), jnp.float32)

    def logits_spec(v):
        return pl.BlockSpec((_TB, v), lambda i: (i, 0))

    mp, zp, md, zd, mv, zv = pl.pallas_call(
        _stats_kernel,
        grid=(_NB,),
        in_specs=[logits_spec(128), logits_spec(512), logits_spec(128)]
                 + [meta_spec] * 3,
        out_specs=[col_spec] * 6,
        out_shape=[col_shape] * 6,
    )(lp, ld, lv, meta(x_t_pitch), meta(x_t_duration), meta(x_t_velocity))

    g_spec = pl.BlockSpec((_TB, 16), lambda i: (i, 0))
    args = [
        gx1p, gx1d, gx1v,
        mp, zp, md, zd, mv, zv,
        meta(x_t_pitch), meta(x_0_pitch), meta(x_1_pitch), meta(log_lambda_pitch),
        meta(x_t_duration), meta(x_0_duration), meta(x_1_duration), meta(log_lambda_duration),
        meta(x_t_velocity), meta(x_0_velocity), meta(x_1_velocity), meta(log_lambda_velocity),
        meta(tl_full),
    ]
    out = pl.pallas_call(
        _combine_kernel,
        grid=(_NB,),
        in_specs=[g_spec] * 3 + [col_spec] * 6 + [meta_spec] * 13,
        out_specs=pl.BlockSpec((1, 1), lambda i: (0, 0)),
        out_shape=jax.ShapeDtypeStruct((1, 1), jnp.float32),
    )(*args)
    return out[0, 0] / _T
